# baseline (speedup 1.0000x reference)
import functools

import jax
import jax.numpy as jnp
from jax import lax
from jax.experimental import pallas as pl
from jax.experimental.pallas import tpu as pltpu
from jax.experimental.pallas import tpu_sc as plsc

_INFO = plsc.get_sparse_core_info()
_NC = _INFO.num_cores
_NS = _INFO.num_subcores
_LANES = _INFO.num_lanes
_NW = _NC * _NS

_K = 10240
_SEG_CHUNK = 8000


def _gather_mul(idx_flat, x_flat, weight):
    n = idx_flat.shape[0]
    w_len = weight.shape[0]
    seg = w_len // _NS
    per_worker = n // _NW
    n_chunks = per_worker // _K
    mesh = plsc.VectorSubcoreMesh(core_axis_name="c", subcore_axis_name="s")

    @functools.partial(
        pl.kernel,
        mesh=mesh,
        out_type=jax.ShapeDtypeStruct((n,), jnp.float32),
        scratch_types=[
            pltpu.VMEM_SHARED((w_len,), jnp.float32),
            pltpu.VMEM((_K,), jnp.int32),
            pltpu.VMEM((_K,), jnp.int32),
            pltpu.VMEM((_K,), jnp.float32),
            pltpu.VMEM((_K,), jnp.float32),
            pltpu.VMEM((_K,), jnp.float32),
            pltpu.VMEM((_K,), jnp.float32),
        ] + [pltpu.SemaphoreType.DMA] * 8,
    )
    def k(idx_hbm, x_hbm, w_hbm, out_hbm, w_sh, idx_v0, idx_v1, w_v0, w_v1,
          x_v0, x_v1, si0, si1, sx0, sx1, sg0, sg1, so0, so1):
        idx_v = (idx_v0, idx_v1)
        w_v = (w_v0, w_v1)
        x_v = (x_v0, x_v1)
        sem_i = (si0, si1)
        sem_x = (sx0, sx1)
        sem_g = (sg0, sg1)
        sem_o = (so0, so1)
        sid = lax.axis_index("s")
        wid = sid * _NC + lax.axis_index("c")
        base = wid * per_worker

        for p in range(seg // _SEG_CHUNK):
            sl = pl.ds(sid * seg + p * _SEG_CHUNK, _SEG_CHUNK)
            pltpu.sync_copy(w_hbm.at[sl], w_v0.at[pl.ds(0, _SEG_CHUNK)])
            pltpu.sync_copy(w_v0.at[pl.ds(0, _SEG_CHUNK)], w_sh.at[sl])
        plsc.subcore_barrier()

        def src(c):
            return pl.ds(base + c * _K, _K)

        h_i, h_x, h_g, h_o = {}, {}, {}, {}

        def stage(c):
            b = c % 2
            h_i[c] = pltpu.async_copy(idx_hbm.at[src(c)], idx_v[b], sem_i[b])
            h_x[c] = pltpu.async_copy(x_hbm.at[src(c)], x_v[b], sem_x[b])

        def fire_gather(c):
            b = c % 2
            h_i[c].wait()
            if c >= 2:
                h_o[c - 2].wait()
            w_src = w_sh if c % 2 == 0 else w_hbm
            h_g[c] = pltpu.async_copy(w_src.at[idx_v[b]], w_v[b], sem_g[b])

        stage(0)
        if n_chunks > 1:
            stage(1)
        fire_gather(0)

        for c in range(n_chunks):
            b = c % 2
            if c + 1 < n_chunks:
                fire_gather(c + 1)

            h_g[c].wait()
            h_x[c].wait()

            wb, xb = w_v[b], x_v[b]

            def body(i, _):
                s = pl.ds(i * _LANES, _LANES)
                wb[s] = wb[s] * xb[s]
                return 0

            lax.fori_loop(0, _K // _LANES, body, 0, unroll=8)

            h_o[c] = pltpu.async_copy(w_v[b], out_hbm.at[src(c)], sem_o[b])
            if c + 2 < n_chunks:
                stage(c + 2)

        h_o[n_chunks - 1].wait()
        if n_chunks > 1:
            h_o[n_chunks - 2].wait()

    return k(idx_flat, x_flat, weight)


def kernel(x, index, weight):
    shape = x.shape
    n = x.size
    idx_flat = index.reshape(n).astype(jnp.int32)
    x_flat = x.reshape(n).astype(jnp.float32)

    tile = _NW * _K
    pad = (-n) % tile
    if pad:
        idx_flat = jnp.pad(idx_flat, (0, pad))
        x_flat = jnp.pad(x_flat, (0, pad))

    w_flat = weight.reshape(weight.size).astype(jnp.float32)
    wpad = (-w_flat.size) % (_NS * _SEG_CHUNK)
    if wpad:
        w_flat = jnp.pad(w_flat, (0, wpad))

    out = _gather_mul(idx_flat, x_flat, w_flat)
    return out[:n].reshape(shape)


# speedup vs baseline: 1.2147x; 1.2147x over previous
import functools

import jax
import jax.numpy as jnp
from jax import lax
from jax.experimental import pallas as pl
from jax.experimental.pallas import tpu as pltpu
from jax.experimental.pallas import tpu_sc as plsc

_INFO = plsc.get_sparse_core_info()
_NC = _INFO.num_cores
_NS = _INFO.num_subcores
_LANES = _INFO.num_lanes
_NW = _NC * _NS

_K = 10240
_SEG_CHUNK = 8000


def _gather_mul(idx_flat, x_flat, weight):
    n = idx_flat.shape[0]
    w_len = weight.shape[0]
    seg = w_len // _NS
    per_worker = n // _NW
    n_chunks = per_worker // _K
    mesh = plsc.VectorSubcoreMesh(core_axis_name="c", subcore_axis_name="s")

    @functools.partial(
        pl.kernel,
        mesh=mesh,
        out_type=jax.ShapeDtypeStruct((n,), jnp.float32),
        scratch_types=[
            pltpu.VMEM_SHARED((w_len,), jnp.float32),
            pltpu.VMEM((_K,), jnp.int32),
            pltpu.VMEM((_K,), jnp.int32),
            pltpu.VMEM((_K,), jnp.float32),
            pltpu.VMEM((_K,), jnp.float32),
            pltpu.VMEM((_K,), jnp.float32),
            pltpu.VMEM((_K,), jnp.float32),
        ] + [pltpu.SemaphoreType.DMA] * 8,
    )
    def k(idx_hbm, x_hbm, w_hbm, out_hbm, w_sh, idx_v0, idx_v1, w_v0, w_v1,
          x_v0, x_v1, si0, si1, sx0, sx1, sg0, sg1, so0, so1):
        idx_v = (idx_v0, idx_v1)
        w_v = (w_v0, w_v1)
        x_v = (x_v0, x_v1)
        sem_i = (si0, si1)
        sem_x = (sx0, sx1)
        sem_g = (sg0, sg1)
        sem_o = (so0, so1)
        sid = lax.axis_index("s")
        wid = sid * _NC + lax.axis_index("c")
        base = wid * per_worker

        for p in range(seg // _SEG_CHUNK):
            sl = pl.ds(sid * seg + p * _SEG_CHUNK, _SEG_CHUNK)
            pltpu.sync_copy(w_hbm.at[sl], w_v0.at[pl.ds(0, _SEG_CHUNK)])
            pltpu.sync_copy(w_v0.at[pl.ds(0, _SEG_CHUNK)], w_sh.at[sl])
        plsc.subcore_barrier()

        def src(c):
            return pl.ds(base + c * _K, _K)

        h_i, h_x, h_g, h_o = {}, {}, {}, {}

        def stage(c):
            b = c % 2
            h_i[c] = pltpu.async_copy(idx_hbm.at[src(c)], idx_v[b], sem_i[b])
            h_x[c] = pltpu.async_copy(x_hbm.at[src(c)], x_v[b], sem_x[b])

        def fire_gather(c):
            b = c % 2
            h_i[c].wait()
            if c >= 2:
                h_o[c - 2].wait()
            h_g[c] = pltpu.async_copy(w_sh.at[idx_v[b]], w_v[b], sem_g[b])

        stage(0)
        if n_chunks > 1:
            stage(1)
        fire_gather(0)

        for c in range(n_chunks):
            b = c % 2
            if c + 1 < n_chunks:
                fire_gather(c + 1)

            h_g[c].wait()
            h_x[c].wait()

            wb, xb = w_v[b], x_v[b]

            def body(i, _):
                s = pl.ds(i * _LANES, _LANES)
                wb[s] = wb[s] * xb[s]
                return 0

            lax.fori_loop(0, _K // _LANES, body, 0, unroll=8)

            h_o[c] = pltpu.async_copy(w_v[b], out_hbm.at[src(c)], sem_o[b])
            if c + 2 < n_chunks:
                stage(c + 2)

        h_o[n_chunks - 1].wait()
        if n_chunks > 1:
            h_o[n_chunks - 2].wait()

    return k(idx_flat, x_flat, weight)


def _i32(a):
    return lax.bitcast_convert_type(a, jnp.int32)


def _f32(a):
    return lax.bitcast_convert_type(a, jnp.float32)


def kernel(x, index, weight):
    shape = x.shape
    n = x.size
    idx_flat = jnp.bitwise_or(index.reshape(n).astype(jnp.int32), 0)
    x_flat = _f32(jnp.bitwise_or(_i32(x.astype(jnp.float32)).reshape(n), 0))

    tile = _NW * _K
    pad = (-n) % tile
    if pad:
        idx_flat = jnp.pad(idx_flat, (0, pad))
        x_flat = jnp.pad(x_flat, (0, pad))

    w_flat = weight.reshape(weight.size).astype(jnp.float32)
    wpad = (-w_flat.size) % (_NS * _SEG_CHUNK)
    if wpad:
        w_flat = jnp.pad(w_flat, (0, wpad))

    out = _gather_mul(idx_flat, x_flat, w_flat)
    return _f32(jnp.bitwise_or(_i32(out[:n]), 0)).reshape(shape)


# speedup vs baseline: 1.2498x; 1.0289x over previous
import functools

import jax
import jax.numpy as jnp
from jax import lax
from jax.experimental import pallas as pl
from jax.experimental.pallas import tpu as pltpu
from jax.experimental.pallas import tpu_sc as plsc

_INFO = plsc.get_sparse_core_info()
_NC = _INFO.num_cores
_NS = _INFO.num_subcores
_LANES = _INFO.num_lanes
_NW = _NC * _NS

_R = 16
_CP = 256
_STAGE = 8192


def _gather_mul(x, idx, weight):
    b, l = x.shape
    w_len = weight.shape[0]
    seg = w_len // _NS
    rows_per_worker = b // _NW
    n_chunks = rows_per_worker // _R
    mesh = plsc.VectorSubcoreMesh(core_axis_name="c", subcore_axis_name="s")

    @functools.partial(
        pl.kernel,
        mesh=mesh,
        out_type=jax.ShapeDtypeStruct((b, l), jnp.float32),
        scratch_types=[
            pltpu.VMEM_SHARED((w_len,), jnp.float32),
            pltpu.VMEM((_STAGE,), jnp.float32),
            pltpu.VMEM((_R, _CP), jnp.int32),
            pltpu.VMEM((_R, _CP), jnp.int32),
            pltpu.VMEM((_R, _CP), jnp.float32),
            pltpu.VMEM((_R, _CP), jnp.float32),
            pltpu.VMEM((_R, _CP), jnp.float32),
            pltpu.VMEM((_R, _CP), jnp.float32),
        ] + [pltpu.SemaphoreType.DMA] * 8,
        compiler_params=pltpu.CompilerParams(use_tc_tiling_on_sc=True),
    )
    def k(x_hbm, idx_hbm, w_hbm, out_hbm, w_sh, stg_v, idx_v0, idx_v1,
          w_v0, w_v1, x_v0, x_v1, si0, si1, sx0, sx1, sg0, sg1, so0, so1):
        idx_v = (idx_v0, idx_v1)
        w_v = (w_v0, w_v1)
        x_v = (x_v0, x_v1)
        sem_i = (si0, si1)
        sem_x = (sx0, sx1)
        sem_g = (sg0, sg1)
        sem_o = (so0, so1)
        sid = lax.axis_index("s")
        wid = sid * _NC + lax.axis_index("c")
        base = wid * rows_per_worker

        for p in range(seg // _STAGE):
            sl = pl.ds(sid * seg + p * _STAGE, _STAGE)
            pltpu.sync_copy(w_hbm.at[sl], stg_v)
            pltpu.sync_copy(stg_v, w_sh.at[sl])
        plsc.subcore_barrier()

        def rows(c):
            return pl.ds(base + c * _R, _R)

        h_i, h_x, h_o = {}, {}, {}

        def stage(c):
            bb = c % 2
            h_i[c] = pltpu.async_copy(idx_hbm.at[rows(c), :], idx_v[bb],
                                      sem_i[bb])
            h_x[c] = pltpu.async_copy(x_hbm.at[rows(c), :], x_v[bb],
                                      sem_x[bb])

        def gather_streams(bb, fn):
            def body(r, _):
                for j in range(_CP // 128):
                    s = pl.ds(j * 128, 128)
                    fn(pltpu.make_async_copy(
                        w_sh.at[idx_v[bb].at[r, s]], w_v[bb].at[r, s],
                        sem_g[bb]))
                return 0

            lax.fori_loop(0, _R, body, 0)

        def fire_gather(c):
            bb = c % 2
            h_i[c].wait()
            if c >= 2:
                h_o[c - 2].wait()
            gather_streams(bb, lambda cp: cp.start())

        stage(0)
        if n_chunks > 1:
            stage(1)
        fire_gather(0)

        for c in range(n_chunks):
            bb = c % 2
            if c + 1 < n_chunks:
                fire_gather(c + 1)

            gather_streams(bb, lambda cp: cp.wait())
            h_x[c].wait()

            wb, xb = w_v[bb], x_v[bb]

            def body(r, _):
                for j in range(_CP // _LANES):
                    s = pl.ds(j * _LANES, _LANES)
                    wb[r, s] = wb[r, s] * xb[r, s]
                return 0

            lax.fori_loop(0, _R, body, 0)

            h_o[c] = pltpu.async_copy(w_v[bb], out_hbm.at[rows(c), :],
                                      sem_o[bb])
            if c + 2 < n_chunks:
                stage(c + 2)

        h_o[n_chunks - 1].wait()
        if n_chunks > 1:
            h_o[n_chunks - 2].wait()

    return k(x, idx, weight)


def kernel(x, index, weight):
    shape = x.shape
    n = x.size
    rows = n // _CP
    x2 = x.astype(jnp.float32).reshape(rows, _CP)
    idx2 = index.astype(jnp.int32).reshape(rows, _CP)

    w_flat = weight.reshape(weight.size).astype(jnp.float32)
    wpad = (-w_flat.size) % (_NS * _STAGE)
    if wpad:
        w_flat = jnp.pad(w_flat, (0, wpad))

    return _gather_mul(x2, idx2, w_flat).reshape(shape)
